# baseline (speedup 1.0000x reference)
import functools

import jax
import jax.numpy as jnp
from jax import lax
from jax.experimental import pallas as pl
from jax.experimental.pallas import tpu as pltpu
from jax.experimental.pallas import tpu_sc as plsc

D = 64
SCALE = 8.0
L = 16

NC, NS = 2, 16
NW = NC * NS

B = 4096 * 200
BPW = B // NW
CH = 512
SUB = 128
NSUB = CH // SUB
NCHUNK = BPW // CH

_mesh = plsc.VectorSubcoreMesh(core_axis_name="c", subcore_axis_name="s")


@functools.partial(
    pl.kernel,
    out_type=jax.ShapeDtypeStruct((B, D), jnp.float32),
    mesh=_mesh,
    scratch_types=[
        pltpu.VMEM((CH,), jnp.int32),
        pltpu.VMEM((CH, D), jnp.float32),
        pltpu.SemaphoreType.DMA,
    ],
    compiler_params=pltpu.CompilerParams(use_tc_tiling_on_sc=False),
)
def _embed_sc(x_hbm, tab_hbm, out_hbm, idx_v, rows_v, sem):
    wid = lax.axis_index("s") * NC + lax.axis_index("c")
    base = wid * BPW
    bias = jnp.full((L,), SCALE, jnp.float32)

    def chunk_body(g, carry):
        off = base + g * CH
        pltpu.sync_copy(x_hbm.at[pl.ds(off, CH)], idx_v)
        copies = [
            pltpu.async_copy(
                tab_hbm.at[idx_v.at[pl.ds(j * SUB, SUB)]],
                rows_v.at[pl.ds(j * SUB, SUB)],
                sem,
            )
            for j in range(NSUB)
        ]
        for c in copies:
            c.wait()

        def add_body(i, c):
            for k in range(D // L):
                sl = pl.ds(k * L, L)
                rows_v[i, sl] = rows_v[i, sl] + bias
            return c

        lax.fori_loop(0, CH, add_body, 0)
        pltpu.sync_copy(rows_v, out_hbm.at[pl.ds(off, CH)])
        return carry

    lax.fori_loop(0, NCHUNK, chunk_body, 0)


def kernel(x, embedding_table):
    out = _embed_sc(x.reshape(B), embedding_table)
    return out.reshape(x.shape + (D,))


# speedup vs baseline: 1.1419x; 1.1419x over previous
import functools

import jax
import jax.numpy as jnp
from jax import lax
from jax.experimental import pallas as pl
from jax.experimental.pallas import tpu as pltpu
from jax.experimental.pallas import tpu_sc as plsc

D = 64
SCALE = 8.0
L = 16

NC, NS = 2, 16
NW = NC * NS

B = 4096 * 200
BPW = B // NW
CH = 512
SUB = 128
NSUB = CH // SUB
NCHUNK = BPW // CH
NPAIR = NCHUNK // 2

_mesh = plsc.VectorSubcoreMesh(core_axis_name="c", subcore_axis_name="s")


@functools.partial(
    pl.kernel,
    out_type=jax.ShapeDtypeStruct((B, D), jnp.float32),
    mesh=_mesh,
    scratch_types=[
        pltpu.VMEM((CH,), jnp.int32),
        pltpu.VMEM((CH,), jnp.int32),
        pltpu.VMEM((CH, D), jnp.float32),
        pltpu.VMEM((CH, D), jnp.float32),
        pltpu.SemaphoreType.DMA,
        pltpu.SemaphoreType.DMA,
        pltpu.SemaphoreType.DMA,
        pltpu.SemaphoreType.DMA,
        pltpu.SemaphoreType.DMA,
        pltpu.SemaphoreType.DMA,
    ],
    compiler_params=pltpu.CompilerParams(use_tc_tiling_on_sc=False),
)
def _embed_sc(x_hbm, tab_hbm, out_hbm, idx0, idx1, rows0, rows1,
              isem0, isem1, gsem0, gsem1, ssem0, ssem1):
    wid = lax.axis_index("s") * NC + lax.axis_index("c")
    base = wid * BPW
    bias = jnp.full((L,), SCALE, jnp.float32)

    idx_v = (idx0, idx1)
    rows_v = (rows0, rows1)
    isem = (isem0, isem1)
    gsem = (gsem0, gsem1)
    ssem = (ssem0, ssem1)

    def fire_idx(g, b):
        pltpu.async_copy(x_hbm.at[pl.ds(base + g * CH, CH)], idx_v[b], isem[b])

    def wait_idx(g, b):
        pltpu.make_async_copy(
            x_hbm.at[pl.ds(base + g * CH, CH)], idx_v[b], isem[b]).wait()

    def fire_gathers(b):
        for j in range(NSUB):
            pltpu.async_copy(
                tab_hbm.at[idx_v[b].at[pl.ds(j * SUB, SUB)]],
                rows_v[b].at[pl.ds(j * SUB, SUB)],
                gsem[b],
            )

    def wait_gathers(b):
        for j in range(NSUB):
            pltpu.make_async_copy(
                tab_hbm.at[idx_v[b].at[pl.ds(j * SUB, SUB)]],
                rows_v[b].at[pl.ds(j * SUB, SUB)],
                gsem[b],
            ).wait()

    def fire_store(g, b):
        pltpu.async_copy(rows_v[b], out_hbm.at[pl.ds(base + g * CH, CH)], ssem[b])

    def wait_store(g, b):
        pltpu.make_async_copy(
            rows_v[b], out_hbm.at[pl.ds(base + g * CH, CH)], ssem[b]).wait()

    def add_bias(b):
        @plsc.parallel_loop(0, CH, unroll=4)
        def _(i):
            for k in range(D // L):
                sl = pl.ds(k * L, L)
                rows_v[b][i, sl] = rows_v[b][i, sl] + bias

    fire_idx(0, 0)
    fire_idx(1, 1)
    wait_idx(0, 0)
    fire_gathers(0)

    def pair_body(p, carry):
        ga = 2 * p
        gb = ga + 1

        wait_idx(gb, 1)

        @pl.when(p > 0)
        def _():
            wait_store(ga - 1, 1)
        fire_gathers(1)

        wait_gathers(0)

        @pl.when(p < NPAIR - 1)
        def _():
            fire_idx(ga + 2, 0)
        add_bias(0)
        fire_store(ga, 0)

        @pl.when(p < NPAIR - 1)
        def _():
            wait_idx(gb + 1, 0)
            wait_store(ga, 0)
            fire_gathers(0)

        wait_gathers(1)

        @pl.when(p < NPAIR - 1)
        def _():
            fire_idx(gb + 2, 1)
        add_bias(1)
        fire_store(gb, 1)
        return carry

    lax.fori_loop(0, NPAIR, pair_body, 0)

    wait_store(NCHUNK - 2, 0)
    wait_store(NCHUNK - 1, 1)


def kernel(x, embedding_table):
    out = _embed_sc(x.reshape(B), embedding_table)
    return out.reshape(x.shape + (D,))
